# baseline (speedup 1.0000x reference)
import jax
import jax.numpy as jnp
from jax import lax
from jax.experimental import pallas as pl
from jax.experimental.pallas import tpu as pltpu
from jax.experimental.pallas import tpu_sc as plsc

N_VOCAB = 100000
NUM_TONES = 16
NUM_LANGUAGES = 10
NCOMBO = NUM_TONES * NUM_LANGUAGES
HIDDEN = 192
B = 64
L = 512
N = B * L
LANES = 16
W = 128
WB = HIDDEN - W
NSLICE_A = W // LANES
NSLICE_B = WB // LANES

NC = 2
NS = 16
NW = NC * NS
ROWS_PER_W = N // NW
CHUNK = 256
NCHUNK = ROWS_PER_W // CHUNK
B_PER_W = B // NW


def _sc_body(idx_hbm, xlen_hbm, emba_hbm, embb_hbm, combo_hbm,
             out_a, out_b, out_m,
             xidx_v, cidx_v, ga_v, gb_v, combo_v, mask_v, xlen_v,
             sema, semb):
    wid = lax.axis_index("s") * NC + lax.axis_index("c")
    wbase = wid * ROWS_PER_W

    pltpu.sync_copy(combo_hbm, combo_v)
    pltpu.sync_copy(xlen_hbm, xlen_v)

    iota = lax.iota(jnp.int32, LANES)

    for i in range(B_PER_W):
        b = wid * B_PER_W + i
        lenvec = plsc.load_gather(xlen_v, [jnp.full((LANES,), b, jnp.int32)])
        for j in range(L // LANES):
            col = iota + (LANES * j)
            m = jnp.where(col < lenvec, jnp.float32(1.0), jnp.float32(0.0))
            mask_v[pl.ds(i * L + LANES * j, LANES)] = m
    pltpu.sync_copy(mask_v, out_m.at[pl.ds(wid * (B_PER_W * L), B_PER_W * L)])

    for c in range(NCHUNK):
        base = wbase + c * CHUNK
        pltpu.sync_copy(idx_hbm.at[pl.ds(base, CHUNK)], xidx_v)
        pltpu.sync_copy(idx_hbm.at[pl.ds(N + base, CHUNK)], cidx_v)

        cp_a = pltpu.async_copy(emba_hbm.at[xidx_v], ga_v, sema)
        cp_b = pltpu.async_copy(embb_hbm.at[xidx_v], gb_v, semb)
        cp_a.wait()
        cp_b.wait()

        def row_body(r, carry):
            rfull = jnp.full((LANES,), r, jnp.int32)
            cv = plsc.load_gather(cidx_v, [rfull]) * HIDDEN
            for j in range(NSLICE_A):
                col = iota + (LANES * j)
                ts = plsc.load_gather(combo_v, [cv + col])
                plsc.addupdate_scatter(ga_v, [rfull, col], ts)
            for j in range(NSLICE_B):
                col = iota + (LANES * j)
                ts = plsc.load_gather(combo_v, [cv + (col + W)])
                plsc.addupdate_scatter(gb_v, [rfull, col + (W - WB)], ts)
            return carry

        lax.fori_loop(0, CHUNK, row_body, 0)
        start = pl.multiple_of(base, CHUNK)
        pltpu.sync_copy(ga_v, out_a.at[pl.ds(start, CHUNK)])
        pltpu.sync_copy(gb_v, out_b.at[pl.ds(start, CHUNK)])


@jax.jit
def _sc_call(idx_cat, xl, emb_a, emb_b, combo):
    mesh = plsc.VectorSubcoreMesh(core_axis_name="c", subcore_axis_name="s")
    return pl.kernel(
        _sc_body,
        out_type=(
            jax.ShapeDtypeStruct((N, W), jnp.float32),
            jax.ShapeDtypeStruct((N, W), jnp.float32),
            jax.ShapeDtypeStruct((B * L,), jnp.float32),
        ),
        mesh=mesh,
        scratch_types=[
            pltpu.VMEM((CHUNK,), jnp.int32),
            pltpu.VMEM((CHUNK,), jnp.int32),
            pltpu.VMEM((CHUNK, W), jnp.float32),
            pltpu.VMEM((CHUNK, W), jnp.float32),
            pltpu.VMEM((NCOMBO * HIDDEN,), jnp.float32),
            pltpu.VMEM((B_PER_W * L,), jnp.float32),
            pltpu.VMEM((B,), jnp.int32),
            pltpu.SemaphoreType.DMA,
            pltpu.SemaphoreType.DMA,
        ],
        compiler_params=pltpu.CompilerParams(
            needs_layout_passes=False, use_tc_tiling_on_sc=True),
    )(idx_cat, xl, emb_a, emb_b, combo)


def kernel(x, x_lengths, tone, language, ja_bert, noise_scale, emb_w, tone_w, lang_w):
    idx_cat = jnp.concatenate([
        x.reshape(-1).astype(jnp.int32),
        tone.reshape(-1).astype(jnp.int32) * NUM_LANGUAGES
        + language.reshape(-1).astype(jnp.int32),
    ])
    xl = x_lengths.astype(jnp.int32)
    emb_a = emb_w[:, :W]
    emb_b = emb_w[:, HIDDEN - W:]
    combo = (tone_w[:, None, :] + lang_w[None, :, :]).reshape(-1)
    h_a, h_b, mask = _sc_call(idx_cat, xl, emb_a, emb_b, combo)
    h = jnp.concatenate([h_a, h_b[:, W - WB:]], axis=1)
    return h.reshape(B, L, HIDDEN), mask.reshape(B, 1, L)


# speedup vs baseline: 1.0302x; 1.0302x over previous
import jax
import jax.numpy as jnp
from jax import lax
from jax.experimental import pallas as pl
from jax.experimental.pallas import tpu as pltpu
from jax.experimental.pallas import tpu_sc as plsc

N_VOCAB = 100000
NUM_TONES = 16
NUM_LANGUAGES = 10
NCOMBO = NUM_TONES * NUM_LANGUAGES
HIDDEN = 192
B = 64
L = 512
N = B * L
LANES = 16
W = 128
WB = HIDDEN - W
NSLICE_A = W // LANES
NSLICE_B = WB // LANES

NC = 2
NS = 16
NW = NC * NS
ROWS_PER_W = N // NW
CHUNK = 256
NCHUNK = ROWS_PER_W // CHUNK
B_PER_W = B // NW


def _sc_body(idx_hbm, xlen_hbm, emba_hbm, embb_hbm, combo_hbm,
             out_a, out_b, out_m,
             xidx_v, cidx_v, ga_v, gb_v, combo_v, mask_v, xlen_v,
             sema, semb):
    wid = lax.axis_index("s") * NC + lax.axis_index("c")
    wbase = wid * ROWS_PER_W

    pltpu.sync_copy(combo_hbm, combo_v)
    pltpu.sync_copy(xlen_hbm, xlen_v)

    iota = lax.iota(jnp.int32, LANES)

    for i in range(B_PER_W):
        b = wid * B_PER_W + i
        lenvec = plsc.load_gather(xlen_v, [jnp.full((LANES,), b, jnp.int32)])
        for j in range(L // LANES):
            col = iota + (LANES * j)
            m = jnp.where(col < lenvec, jnp.float32(1.0), jnp.float32(0.0))
            mask_v[pl.ds(i * L + LANES * j, LANES)] = m
    pltpu.sync_copy(mask_v, out_m.at[pl.ds(wid * (B_PER_W * L), B_PER_W * L)])

    for c in range(NCHUNK):
        base = wbase + c * CHUNK
        pltpu.sync_copy(idx_hbm.at[pl.ds(base, CHUNK)], xidx_v)
        pltpu.sync_copy(idx_hbm.at[pl.ds(N + base, CHUNK)], cidx_v)

        cp_a = pltpu.async_copy(emba_hbm.at[xidx_v], ga_v, sema)
        cp_b = pltpu.async_copy(embb_hbm.at[xidx_v], gb_v, semb)
        cp_a.wait()
        cp_b.wait()

        def row_body(r, carry):
            rfull = jnp.full((LANES,), r, jnp.int32)
            cv = plsc.load_gather(cidx_v, [rfull]) * HIDDEN
            for j in range(NSLICE_A):
                col = iota + (LANES * j)
                ts = plsc.load_gather(combo_v, [cv + col])
                plsc.addupdate_scatter(ga_v, [rfull, col], ts)
            for j in range(NSLICE_B):
                col = iota + (LANES * j)
                ts = plsc.load_gather(combo_v, [cv + (col + W)])
                plsc.addupdate_scatter(gb_v, [rfull, col], ts)
            return carry

        lax.fori_loop(0, CHUNK, row_body, 0)
        start = pl.multiple_of(base, CHUNK)
        pltpu.sync_copy(ga_v, out_a.at[pl.ds(start, CHUNK)])
        pltpu.sync_copy(gb_v, out_b.at[pl.ds(start, CHUNK)])


@jax.jit
def _sc_call(idx_cat, xl, emb_a, emb_b, combo):
    mesh = plsc.VectorSubcoreMesh(core_axis_name="c", subcore_axis_name="s")
    return pl.kernel(
        _sc_body,
        out_type=(
            jax.ShapeDtypeStruct((N, W), jnp.float32),
            jax.ShapeDtypeStruct((N, W), jnp.float32),
            jax.ShapeDtypeStruct((B * L,), jnp.float32),
        ),
        mesh=mesh,
        scratch_types=[
            pltpu.VMEM((CHUNK,), jnp.int32),
            pltpu.VMEM((CHUNK,), jnp.int32),
            pltpu.VMEM((CHUNK, W), jnp.float32),
            pltpu.VMEM((CHUNK, W), jnp.float32),
            pltpu.VMEM((NCOMBO * HIDDEN,), jnp.float32),
            pltpu.VMEM((B_PER_W * L,), jnp.float32),
            pltpu.VMEM((B,), jnp.int32),
            pltpu.SemaphoreType.DMA,
            pltpu.SemaphoreType.DMA,
        ],
        compiler_params=pltpu.CompilerParams(
            needs_layout_passes=False, use_tc_tiling_on_sc=True),
    )(idx_cat, xl, emb_a, emb_b, combo)


def kernel(x, x_lengths, tone, language, ja_bert, noise_scale, emb_w, tone_w, lang_w):
    idx_cat = jnp.concatenate([
        x.reshape(-1).astype(jnp.int32),
        tone.reshape(-1).astype(jnp.int32) * NUM_LANGUAGES
        + language.reshape(-1).astype(jnp.int32),
    ])
    xl = x_lengths.astype(jnp.int32)
    emb_a = emb_w[:, :W]
    emb_b = jnp.pad(emb_w[:, W:], ((0, 0), (0, W - WB)))
    combo = (tone_w[:, None, :] + lang_w[None, :, :]).reshape(-1)
    h_a, h_b, mask = _sc_call(idx_cat, xl, emb_a, emb_b, combo)
    h = jnp.concatenate([h_a, h_b[:, :WB]], axis=1)
    return h.reshape(B, L, HIDDEN), mask.reshape(B, 1, L)
